# baseline (speedup 1.0000x reference)
import functools

import jax
import jax.numpy as jnp
from jax import lax
from jax.experimental import pallas as pl
from jax.experimental.pallas import tpu as pltpu

N = 10000
E = 320000
H = 128
OUT = 64
G = 16

NTILES = 32
RANGE = 320
NPAD = NTILES * RANGE
E2 = NTILES * 10240
BE = 2048

NEG_INF = float("-inf")


def _dot(a, b):
    return lax.dot(a, b)


def _enc_body(h_ref, W_ref, b_ref, o_ref):
    o_ref[...] = _dot(h_ref[...], W_ref[...]) + b_ref[0, :]


def _encode(h, W_enc, b_enc):
    return pl.pallas_call(
        _enc_body,
        out_shape=jax.ShapeDtypeStruct((N, H), jnp.float32),
    )(h, W_enc, b_enc.reshape(1, H))


def _node_pre_body(x_ref, pos_ref, hW1_ref, hb1_ref, hW2_ref, hb2_ref,
                   Wx_ref, S_ref, DP_ref):
    x = x_ref[...]
    pos = pos_ref[...]
    hid = jnp.maximum(_dot(x, hW1_ref[...]) + hb1_ref[0, :], 0.0)
    delta = _dot(hid, hW2_ref[...]) + hb2_ref[0, :]
    S_ref[...] = _dot(x, Wx_ref[...])
    DP_ref[...] = delta - pos


def _node_pre(x_pad, pos16, hW1, hb1, hW2, hb2, fW1):
    Wx = fW1[3:]
    hW2_16 = jnp.pad(hW2, ((0, 0), (0, 13)))
    hb2_16 = jnp.pad(hb2.reshape(1, 3), ((0, 0), (0, 13)))
    BN = 2048
    fixed = lambda i: (0, 0)
    return pl.pallas_call(
        _node_pre_body,
        grid=(NPAD // BN,),
        in_specs=[
            pl.BlockSpec((BN, H), lambda i: (i, 0)),
            pl.BlockSpec((BN, 16), lambda i: (i, 0)),
            pl.BlockSpec((H, H), fixed),
            pl.BlockSpec((1, H), fixed),
            pl.BlockSpec((H, 16), fixed),
            pl.BlockSpec((1, 16), fixed),
            pl.BlockSpec((H, H), fixed),
        ],
        out_specs=(pl.BlockSpec((BN, H), lambda i: (i, 0)),
                   pl.BlockSpec((BN, 16), lambda i: (i, 0))),
        out_shape=(jax.ShapeDtypeStruct((NPAD, H), jnp.float32),
                   jax.ShapeDtypeStruct((NPAD, 16), jnp.float32)),
    )(x_pad, pos16, hW1, hb1.reshape(1, H), hW2_16, hb2_16, Wx)


def _edge_mlp_body(u1_ref, p1_ref, p2_ref, Wp_ref, b1_ref, W2_ref, b2_ref,
                   o_ref):
    pd = p1_ref[...] + p2_ref[...]
    pre = _dot(pd, Wp_ref[...]) + u1_ref[...] + b1_ref[0, :]
    hid = jnp.maximum(pre, 0.0)
    o_ref[...] = _dot(hid, W2_ref[...]) + b2_ref[0, :]


def _edge_mlp(U1, P1, P2, fW1, fb1, fW2, fb2):
    Wp16 = jnp.pad(fW1[:3], ((0, 13), (0, 0)))
    nblk = E2 // BE
    fixed = lambda i: (0, 0)
    return pl.pallas_call(
        _edge_mlp_body,
        grid=(nblk,),
        in_specs=[
            pl.BlockSpec((BE, H), lambda i: (i, 0)),
            pl.BlockSpec((BE, 16), lambda i: (i, 0)),
            pl.BlockSpec((BE, 16), lambda i: (i, 0)),
            pl.BlockSpec((16, H), fixed),
            pl.BlockSpec((1, H), fixed),
            pl.BlockSpec((H, H), fixed),
            pl.BlockSpec((1, H), fixed),
        ],
        out_specs=pl.BlockSpec((BE, H), lambda i: (i, 0)),
        out_shape=jax.ShapeDtypeStruct((E2, H), jnp.float32),
    )(U1, P1, P2, Wp16, fb1.reshape(1, H), fW2, fb2.reshape(1, H))


def _post_body(x_ref, agg_ref, gW1_ref, gb1_ref, gam_ref, bet_ref,
               gW2_ref, gb2_ref, o_ref):
    agg = agg_ref[...]
    agg = jnp.where(agg == NEG_INF, 0.0, agg)
    t = jnp.maximum(_dot(agg, gW1_ref[...]) + gb1_ref[0, :], 0.0)
    m = jnp.mean(t, axis=0)
    v = jnp.mean((t - m[None, :]) ** 2, axis=0)
    t = (t - m[None, :]) / jnp.sqrt(v + 1e-5) * gam_ref[0, :] + bet_ref[0, :]
    o_ref[...] = x_ref[...] + _dot(t, gW2_ref[...]) + gb2_ref[0, :]


def _post(x, agg, gW1, gb1, gamma, beta, gW2, gb2):
    return pl.pallas_call(
        _post_body,
        out_shape=jax.ShapeDtypeStruct((N, H), jnp.float32),
    )(x, agg, gW1, gb1.reshape(1, H), gamma.reshape(1, H),
      beta.reshape(1, H), gW2, gb2.reshape(1, H))


def _pool_head_body(x_ref, batch_ref, rW1_ref, rb1_ref, rW2_ref, rb2_ref,
                    out_ref):
    x = x_ref[...]
    batch = batch_ref[0, :]
    seg = lax.broadcasted_iota(jnp.int32, (G, x.shape[0]), 0)
    onehot = (seg == batch[None, :]).astype(jnp.float32)
    sums = _dot(onehot, x)
    counts = jnp.sum(onehot, axis=1)
    pooled = sums / jnp.maximum(counts, 1.0)[:, None]
    hid = jnp.maximum(_dot(pooled, rW1_ref[...]) + rb1_ref[0, :], 0.0)
    out_ref[...] = _dot(hid, rW2_ref[...]) + rb2_ref[0, :]


def _pool_head(x, batch, r_W1, r_b1, r_W2, r_b2):
    return pl.pallas_call(
        _pool_head_body,
        out_shape=jax.ShapeDtypeStruct((G, OUT), jnp.float32),
    )(x, batch.reshape(1, N), r_W1, r_b1.reshape(1, H),
      r_W2, r_b2.reshape(1, OUT))


def _edge_gather(S, pos16, DP16, src2, dst2):
    return (jnp.take(S, src2, axis=0), jnp.take(pos16, src2, axis=0),
            jnp.take(DP16, dst2, axis=0))


def _seg_max(M, dst2):
    return jax.ops.segment_max(M, dst2, num_segments=NPAD)


def _conv(x, pos16, src2, dst2, hW1, hb1, hW2, hb2, fW1, fb1, fW2, fb2,
          gW1, gb1, gamma, beta, gW2, gb2):
    x_pad = jnp.pad(x, ((0, NPAD - N), (0, 0)))
    S, DP16 = _node_pre(x_pad, pos16, hW1, hb1, hW2, hb2, fW1)
    U1, P1, P2 = _edge_gather(S, pos16, DP16, src2, dst2)
    M = _edge_mlp(U1, P1, P2, fW1, fb1, fW2, fb2)
    agg = _seg_max(M, dst2)
    return _post(x, agg[:N], gW1, gb1, gamma, beta, gW2, gb2)


def kernel(h, pos, edge_index, batch, W_enc, b_enc, h1_W1, h1_b1, h1_W2,
           h1_b2, f1_W1, f1_b1, f1_W2, f1_b2, g1_W1, g1_b1, g1_gamma,
           g1_beta, g1_W2, g1_b2, h2_W1, h2_b1, h2_W2, h2_b2, f2_W1, f2_b1,
           f2_W2, f2_b2, g2_W1, g2_b1, g2_gamma, g2_beta, g2_W2, g2_b2,
           r_W1, r_b1, r_W2, r_b2):
    src2 = jnp.concatenate(
        [edge_index[0], jnp.zeros((E2 - E,), jnp.int32)])
    dst2 = jnp.concatenate(
        [edge_index[1], jnp.full((E2 - E,), N, jnp.int32)])
    pos16 = jnp.pad(pos, ((0, NPAD - N), (0, 13)))

    x = _encode(h, W_enc, b_enc)
    x = _conv(x, pos16, src2, dst2, h1_W1, h1_b1, h1_W2, h1_b2,
              f1_W1, f1_b1, f1_W2, f1_b2, g1_W1, g1_b1, g1_gamma, g1_beta,
              g1_W2, g1_b2)
    x = _conv(x, pos16, src2, dst2, h2_W1, h2_b1, h2_W2, h2_b2,
              f2_W1, f2_b1, f2_W2, f2_b2, g2_W1, g2_b1, g2_gamma, g2_beta,
              g2_W2, g2_b2)
    return _pool_head(x, batch, r_W1, r_b1, r_W2, r_b2)


# speedup vs baseline: 2.2318x; 2.2318x over previous
import functools

import jax
import jax.numpy as jnp
from jax import lax
from jax.experimental import pallas as pl
from jax.experimental.pallas import tpu as pltpu
from jax.experimental.pallas import tpu_sc as plsc

N = 10000
E = 320000
H = 128
OUT = 64
G = 16

NTILES = 32
RANGE = 320
NPAD = NTILES * RANGE
E2 = NTILES * 10240
BE = 2048

NEG_INF = float("-inf")


def _dot(a, b):
    return lax.dot(a, b)


def _enc_body(h_ref, W_ref, b_ref, o_ref):
    o_ref[...] = _dot(h_ref[...], W_ref[...]) + b_ref[0, :]


def _encode(h, W_enc, b_enc):
    return pl.pallas_call(
        _enc_body,
        out_shape=jax.ShapeDtypeStruct((N, H), jnp.float32),
    )(h, W_enc, b_enc.reshape(1, H))


def _node_pre_body(x_ref, pos_ref, hW1_ref, hb1_ref, hW2_ref, hb2_ref,
                   Wx_ref, S_ref, DP_ref):
    x = x_ref[...]
    pos = pos_ref[...]
    hid = jnp.maximum(_dot(x, hW1_ref[...]) + hb1_ref[0, :], 0.0)
    delta = _dot(hid, hW2_ref[...]) + hb2_ref[0, :]
    S_ref[...] = _dot(x, Wx_ref[...])
    DP_ref[...] = delta - pos


def _node_pre(x_pad, pos16, hW1, hb1, hW2, hb2, fW1):
    Wx = fW1[3:]
    hW2_16 = jnp.pad(hW2, ((0, 0), (0, 13)))
    hb2_16 = jnp.pad(hb2.reshape(1, 3), ((0, 0), (0, 13)))
    BN = 2048
    fixed = lambda i: (0, 0)
    return pl.pallas_call(
        _node_pre_body,
        grid=(NPAD // BN,),
        in_specs=[
            pl.BlockSpec((BN, H), lambda i: (i, 0)),
            pl.BlockSpec((BN, 16), lambda i: (i, 0)),
            pl.BlockSpec((H, H), fixed),
            pl.BlockSpec((1, H), fixed),
            pl.BlockSpec((H, 16), fixed),
            pl.BlockSpec((1, 16), fixed),
            pl.BlockSpec((H, H), fixed),
        ],
        out_specs=(pl.BlockSpec((BN, H), lambda i: (i, 0)),
                   pl.BlockSpec((BN, 16), lambda i: (i, 0))),
        out_shape=(jax.ShapeDtypeStruct((NPAD, H), jnp.float32),
                   jax.ShapeDtypeStruct((NPAD, 16), jnp.float32)),
    )(x_pad, pos16, hW1, hb1.reshape(1, H), hW2_16, hb2_16, Wx)


def _edge_mlp_body(u1_ref, p1_ref, p2_ref, Wp_ref, b1_ref, W2_ref, b2_ref,
                   o_ref):
    pd = p1_ref[...] + p2_ref[...]
    pre = _dot(pd, Wp_ref[...]) + u1_ref[...] + b1_ref[0, :]
    hid = jnp.maximum(pre, 0.0)
    o_ref[...] = _dot(hid, W2_ref[...]) + b2_ref[0, :]


def _edge_mlp(U1, P1, P2, fW1, fb1, fW2, fb2):
    Wp16 = jnp.pad(fW1[:3], ((0, 13), (0, 0)))
    nblk = E2 // BE
    fixed = lambda i: (0, 0)
    return pl.pallas_call(
        _edge_mlp_body,
        grid=(nblk,),
        in_specs=[
            pl.BlockSpec((BE, H), lambda i: (i, 0)),
            pl.BlockSpec((BE, 16), lambda i: (i, 0)),
            pl.BlockSpec((BE, 16), lambda i: (i, 0)),
            pl.BlockSpec((16, H), fixed),
            pl.BlockSpec((1, H), fixed),
            pl.BlockSpec((H, H), fixed),
            pl.BlockSpec((1, H), fixed),
        ],
        out_specs=pl.BlockSpec((BE, H), lambda i: (i, 0)),
        out_shape=jax.ShapeDtypeStruct((E2, H), jnp.float32),
    )(U1, P1, P2, Wp16, fb1.reshape(1, H), fW2, fb2.reshape(1, H))


def _post_body(x_ref, agg_ref, gW1_ref, gb1_ref, gam_ref, bet_ref,
               gW2_ref, gb2_ref, o_ref):
    agg = agg_ref[...]
    agg = jnp.where(agg == NEG_INF, 0.0, agg)
    t = jnp.maximum(_dot(agg, gW1_ref[...]) + gb1_ref[0, :], 0.0)
    m = jnp.mean(t, axis=0)
    v = jnp.mean((t - m[None, :]) ** 2, axis=0)
    t = (t - m[None, :]) / jnp.sqrt(v + 1e-5) * gam_ref[0, :] + bet_ref[0, :]
    o_ref[...] = x_ref[...] + _dot(t, gW2_ref[...]) + gb2_ref[0, :]


def _post(x, agg, gW1, gb1, gamma, beta, gW2, gb2):
    return pl.pallas_call(
        _post_body,
        out_shape=jax.ShapeDtypeStruct((N, H), jnp.float32),
    )(x, agg, gW1, gb1.reshape(1, H), gamma.reshape(1, H),
      beta.reshape(1, H), gW2, gb2.reshape(1, H))


def _pool_head_body(x_ref, batch_ref, rW1_ref, rb1_ref, rW2_ref, rb2_ref,
                    out_ref):
    x = x_ref[...]
    batch = batch_ref[0, :]
    seg = lax.broadcasted_iota(jnp.int32, (G, x.shape[0]), 0)
    onehot = (seg == batch[None, :]).astype(jnp.float32)
    sums = _dot(onehot, x)
    counts = jnp.sum(onehot, axis=1)
    pooled = sums / jnp.maximum(counts, 1.0)[:, None]
    hid = jnp.maximum(_dot(pooled, rW1_ref[...]) + rb1_ref[0, :], 0.0)
    out_ref[...] = _dot(hid, rW2_ref[...]) + rb2_ref[0, :]


def _pool_head(x, batch, r_W1, r_b1, r_W2, r_b2):
    return pl.pallas_call(
        _pool_head_body,
        out_shape=jax.ShapeDtypeStruct((G, OUT), jnp.float32),
    )(x, batch.reshape(1, N), r_W1, r_b1.reshape(1, H),
      r_W2, r_b2.reshape(1, OUT))


EPT = E2 // NTILES
GK = 128


def _gather_body(S_hbm, pos_hbm, dp_hbm, src_hbm, dst_hbm,
                 u1_hbm, p1_hbm, p2_hbm,
                 sidx, didx, sbuf, p1buf, p2buf, sem):
    wid = lax.axis_index("s") * 2 + lax.axis_index("c")
    base = wid * EPT

    def body(i, carry):
        off = base + i * GK
        pltpu.sync_copy(src_hbm.at[pl.ds(off, GK)], sidx)
        pltpu.sync_copy(dst_hbm.at[pl.ds(off, GK)], didx)
        c1 = pltpu.async_copy(S_hbm.at[sidx], sbuf, sem)
        c2 = pltpu.async_copy(pos_hbm.at[sidx], p1buf, sem)
        c3 = pltpu.async_copy(dp_hbm.at[didx], p2buf, sem)
        c1.wait()
        c2.wait()
        c3.wait()
        pltpu.sync_copy(sbuf, u1_hbm.at[pl.ds(off, GK)])
        pltpu.sync_copy(p1buf, p1_hbm.at[pl.ds(off, GK)])
        pltpu.sync_copy(p2buf, p2_hbm.at[pl.ds(off, GK)])
        return carry

    lax.fori_loop(0, EPT // GK, body, 0)


def _edge_gather(S, pos16, DP16, src2, dst2):
    mesh = plsc.VectorSubcoreMesh(core_axis_name="c", subcore_axis_name="s")
    f = functools.partial(
        pl.kernel,
        mesh=mesh,
        compiler_params=pltpu.CompilerParams(use_tc_tiling_on_sc=False),
        out_type=(jax.ShapeDtypeStruct((E2, H), jnp.float32),
                  jax.ShapeDtypeStruct((E2, 16), jnp.float32),
                  jax.ShapeDtypeStruct((E2, 16), jnp.float32)),
        scratch_types=[
            pltpu.VMEM((GK,), jnp.int32),
            pltpu.VMEM((GK,), jnp.int32),
            pltpu.VMEM((GK, H), jnp.float32),
            pltpu.VMEM((GK, 16), jnp.float32),
            pltpu.VMEM((GK, 16), jnp.float32),
            pltpu.SemaphoreType.DMA,
        ],
    )(_gather_body)
    return f(S, pos16, DP16, src2, dst2)


def _seg_max(M, dst2):
    return jax.ops.segment_max(M, dst2, num_segments=NPAD)


def _conv(x, pos16, src2, dst2, hW1, hb1, hW2, hb2, fW1, fb1, fW2, fb2,
          gW1, gb1, gamma, beta, gW2, gb2):
    x_pad = jnp.pad(x, ((0, NPAD - N), (0, 0)))
    S, DP16 = _node_pre(x_pad, pos16, hW1, hb1, hW2, hb2, fW1)
    U1, P1, P2 = _edge_gather(S, pos16, DP16, src2, dst2)
    M = _edge_mlp(U1, P1, P2, fW1, fb1, fW2, fb2)
    agg = _seg_max(M, dst2)
    return _post(x, agg[:N], gW1, gb1, gamma, beta, gW2, gb2)


def kernel(h, pos, edge_index, batch, W_enc, b_enc, h1_W1, h1_b1, h1_W2,
           h1_b2, f1_W1, f1_b1, f1_W2, f1_b2, g1_W1, g1_b1, g1_gamma,
           g1_beta, g1_W2, g1_b2, h2_W1, h2_b1, h2_W2, h2_b2, f2_W1, f2_b1,
           f2_W2, f2_b2, g2_W1, g2_b1, g2_gamma, g2_beta, g2_W2, g2_b2,
           r_W1, r_b1, r_W2, r_b2):
    src2 = jnp.concatenate(
        [edge_index[0], jnp.zeros((E2 - E,), jnp.int32)])
    dst2 = jnp.concatenate(
        [edge_index[1], jnp.full((E2 - E,), N, jnp.int32)])
    pos16 = jnp.pad(pos, ((0, NPAD - N), (0, 13)))

    x = _encode(h, W_enc, b_enc)
    x = _conv(x, pos16, src2, dst2, h1_W1, h1_b1, h1_W2, h1_b2,
              f1_W1, f1_b1, f1_W2, f1_b2, g1_W1, g1_b1, g1_gamma, g1_beta,
              g1_W2, g1_b2)
    x = _conv(x, pos16, src2, dst2, h2_W1, h2_b1, h2_W2, h2_b2,
              f2_W1, f2_b1, f2_W2, f2_b2, g2_W1, g2_b1, g2_gamma, g2_beta,
              g2_W2, g2_b2)
    return _pool_head(x, batch, r_W1, r_b1, r_W2, r_b2)


# speedup vs baseline: 2.3036x; 1.0322x over previous
import functools

import jax
import jax.numpy as jnp
from jax import lax
from jax.experimental import pallas as pl
from jax.experimental.pallas import tpu as pltpu
from jax.experimental.pallas import tpu_sc as plsc

N = 10000
E = 320000
H = 128
OUT = 64
G = 16

NTILES = 32
RANGE = 320
NPAD = NTILES * RANGE
E2 = NTILES * 10240
BE = 2048

NEG_INF = float("-inf")


def _dot(a, b):
    return lax.dot(a, b)


def _enc_body(h_ref, W_ref, b_ref, o_ref):
    o_ref[...] = _dot(h_ref[...], W_ref[...]) + b_ref[0, :]


def _encode(h, W_enc, b_enc):
    return pl.pallas_call(
        _enc_body,
        out_shape=jax.ShapeDtypeStruct((N, H), jnp.float32),
    )(h, W_enc, b_enc.reshape(1, H))


def _node_pre_body(x_ref, pos_ref, hW1_ref, hb1_ref, hW2_ref, hb2_ref,
                   Wx_ref, S_ref, DP_ref):
    x = x_ref[...]
    pos = pos_ref[...]
    hid = jnp.maximum(_dot(x, hW1_ref[...]) + hb1_ref[0, :], 0.0)
    delta = _dot(hid, hW2_ref[...]) + hb2_ref[0, :]
    S_ref[...] = _dot(x, Wx_ref[...])
    DP_ref[...] = delta - pos


def _node_pre(x_pad, pos16, hW1, hb1, hW2, hb2, fW1):
    Wx = fW1[3:]
    hW2_16 = jnp.pad(hW2, ((0, 0), (0, 13)))
    hb2_16 = jnp.pad(hb2.reshape(1, 3), ((0, 0), (0, 13)))
    BN = 2048
    fixed = lambda i: (0, 0)
    return pl.pallas_call(
        _node_pre_body,
        grid=(NPAD // BN,),
        in_specs=[
            pl.BlockSpec((BN, H), lambda i: (i, 0)),
            pl.BlockSpec((BN, 16), lambda i: (i, 0)),
            pl.BlockSpec((H, H), fixed),
            pl.BlockSpec((1, H), fixed),
            pl.BlockSpec((H, 16), fixed),
            pl.BlockSpec((1, 16), fixed),
            pl.BlockSpec((H, H), fixed),
        ],
        out_specs=(pl.BlockSpec((BN, H), lambda i: (i, 0)),
                   pl.BlockSpec((BN, 16), lambda i: (i, 0))),
        out_shape=(jax.ShapeDtypeStruct((NPAD, H), jnp.float32),
                   jax.ShapeDtypeStruct((NPAD, 16), jnp.float32)),
    )(x_pad, pos16, hW1, hb1.reshape(1, H), hW2_16, hb2_16, Wx)


def _edge_mlp_body(u1_ref, p1_ref, p2_ref, Wp_ref, b1_ref, W2_ref, b2_ref,
                   o_ref):
    pd = p1_ref[...] + p2_ref[...]
    pre = _dot(pd, Wp_ref[...]) + u1_ref[...] + b1_ref[0, :]
    hid = jnp.maximum(pre, 0.0)
    o_ref[...] = _dot(hid, W2_ref[...]) + b2_ref[0, :]


def _edge_mlp(U1, P1, P2, fW1, fb1, fW2, fb2):
    Wp16 = jnp.pad(fW1[:3], ((0, 13), (0, 0)))
    nblk = E2 // BE
    fixed = lambda i: (0, 0)
    return pl.pallas_call(
        _edge_mlp_body,
        grid=(nblk,),
        in_specs=[
            pl.BlockSpec((BE, H), lambda i: (i, 0)),
            pl.BlockSpec((BE, 16), lambda i: (i, 0)),
            pl.BlockSpec((BE, 16), lambda i: (i, 0)),
            pl.BlockSpec((16, H), fixed),
            pl.BlockSpec((1, H), fixed),
            pl.BlockSpec((H, H), fixed),
            pl.BlockSpec((1, H), fixed),
        ],
        out_specs=pl.BlockSpec((BE, H), lambda i: (i, 0)),
        out_shape=jax.ShapeDtypeStruct((E2, H), jnp.float32),
    )(U1, P1, P2, Wp16, fb1.reshape(1, H), fW2, fb2.reshape(1, H))


def _post_body(x_ref, agg_ref, gW1_ref, gb1_ref, gam_ref, bet_ref,
               gW2_ref, gb2_ref, o_ref):
    agg = agg_ref[...]
    agg = jnp.where(agg == NEG_INF, 0.0, agg)
    t = jnp.maximum(_dot(agg, gW1_ref[...]) + gb1_ref[0, :], 0.0)
    m = jnp.mean(t, axis=0)
    v = jnp.mean((t - m[None, :]) ** 2, axis=0)
    t = (t - m[None, :]) / jnp.sqrt(v + 1e-5) * gam_ref[0, :] + bet_ref[0, :]
    o_ref[...] = x_ref[...] + _dot(t, gW2_ref[...]) + gb2_ref[0, :]


def _post(x, agg, gW1, gb1, gamma, beta, gW2, gb2):
    return pl.pallas_call(
        _post_body,
        out_shape=jax.ShapeDtypeStruct((N, H), jnp.float32),
    )(x, agg, gW1, gb1.reshape(1, H), gamma.reshape(1, H),
      beta.reshape(1, H), gW2, gb2.reshape(1, H))


def _pool_head_body(x_ref, batch_ref, rW1_ref, rb1_ref, rW2_ref, rb2_ref,
                    out_ref):
    x = x_ref[...]
    batch = batch_ref[0, :]
    seg = lax.broadcasted_iota(jnp.int32, (G, x.shape[0]), 0)
    onehot = (seg == batch[None, :]).astype(jnp.float32)
    sums = _dot(onehot, x)
    counts = jnp.sum(onehot, axis=1)
    pooled = sums / jnp.maximum(counts, 1.0)[:, None]
    hid = jnp.maximum(_dot(pooled, rW1_ref[...]) + rb1_ref[0, :], 0.0)
    out_ref[...] = _dot(hid, rW2_ref[...]) + rb2_ref[0, :]


def _pool_head(x, batch, r_W1, r_b1, r_W2, r_b2):
    return pl.pallas_call(
        _pool_head_body,
        out_shape=jax.ShapeDtypeStruct((G, OUT), jnp.float32),
    )(x, batch.reshape(1, N), r_W1, r_b1.reshape(1, H),
      r_W2, r_b2.reshape(1, OUT))


EPT = E2 // NTILES
GK = 128


def _gather_body(S_hbm, pos_hbm, dp_hbm, src_hbm, dst_hbm,
                 u1_hbm, p1_hbm, p2_hbm,
                 sidx0, didx0, sbuf0, p1buf0, p2buf0,
                 sidx1, didx1, sbuf1, p1buf1, p2buf1, sem):
    wid = lax.axis_index("s") * 2 + lax.axis_index("c")
    base = wid * EPT
    bufs = ((sidx0, didx0, sbuf0, p1buf0, p2buf0),
            (sidx1, didx1, sbuf1, p1buf1, p2buf1))

    def body(i, carry):
        offs = (base + (2 * i) * GK, base + (2 * i + 1) * GK)
        copies = []
        for b in range(2):
            sidx, didx, sbuf, p1buf, p2buf = bufs[b]
            off = offs[b]
            pltpu.sync_copy(src_hbm.at[pl.ds(off, GK)], sidx)
            pltpu.sync_copy(dst_hbm.at[pl.ds(off, GK)], didx)
            copies.append((pltpu.async_copy(S_hbm.at[sidx], sbuf, sem),
                           pltpu.async_copy(pos_hbm.at[sidx], p1buf, sem),
                           pltpu.async_copy(dp_hbm.at[didx], p2buf, sem)))
        for b in range(2):
            sidx, didx, sbuf, p1buf, p2buf = bufs[b]
            off = offs[b]
            for c in copies[b]:
                c.wait()
            pltpu.sync_copy(sbuf, u1_hbm.at[pl.ds(off, GK)])
            pltpu.sync_copy(p1buf, p1_hbm.at[pl.ds(off, GK)])
            pltpu.sync_copy(p2buf, p2_hbm.at[pl.ds(off, GK)])
        return carry

    lax.fori_loop(0, EPT // GK // 2, body, 0)


def _edge_gather(S, pos16, DP16, src2, dst2):
    mesh = plsc.VectorSubcoreMesh(core_axis_name="c", subcore_axis_name="s")
    f = functools.partial(
        pl.kernel,
        mesh=mesh,
        compiler_params=pltpu.CompilerParams(use_tc_tiling_on_sc=False),
        out_type=(jax.ShapeDtypeStruct((E2, H), jnp.float32),
                  jax.ShapeDtypeStruct((E2, 16), jnp.float32),
                  jax.ShapeDtypeStruct((E2, 16), jnp.float32)),
        scratch_types=[
            pltpu.VMEM((GK,), jnp.int32),
            pltpu.VMEM((GK,), jnp.int32),
            pltpu.VMEM((GK, H), jnp.float32),
            pltpu.VMEM((GK, 16), jnp.float32),
            pltpu.VMEM((GK, 16), jnp.float32),
            pltpu.VMEM((GK,), jnp.int32),
            pltpu.VMEM((GK,), jnp.int32),
            pltpu.VMEM((GK, H), jnp.float32),
            pltpu.VMEM((GK, 16), jnp.float32),
            pltpu.VMEM((GK, 16), jnp.float32),
            pltpu.SemaphoreType.DMA,
        ],
    )(_gather_body)
    return f(S, pos16, DP16, src2, dst2)


def _seg_max(M, dst2):
    return jax.ops.segment_max(M, dst2, num_segments=NPAD)


def _conv(x, pos16, src2, dst2, hW1, hb1, hW2, hb2, fW1, fb1, fW2, fb2,
          gW1, gb1, gamma, beta, gW2, gb2):
    x_pad = jnp.pad(x, ((0, NPAD - N), (0, 0)))
    S, DP16 = _node_pre(x_pad, pos16, hW1, hb1, hW2, hb2, fW1)
    U1, P1, P2 = _edge_gather(S, pos16, DP16, src2, dst2)
    M = _edge_mlp(U1, P1, P2, fW1, fb1, fW2, fb2)
    agg = _seg_max(M, dst2)
    return _post(x, agg[:N], gW1, gb1, gamma, beta, gW2, gb2)


def kernel(h, pos, edge_index, batch, W_enc, b_enc, h1_W1, h1_b1, h1_W2,
           h1_b2, f1_W1, f1_b1, f1_W2, f1_b2, g1_W1, g1_b1, g1_gamma,
           g1_beta, g1_W2, g1_b2, h2_W1, h2_b1, h2_W2, h2_b2, f2_W1, f2_b1,
           f2_W2, f2_b2, g2_W1, g2_b1, g2_gamma, g2_beta, g2_W2, g2_b2,
           r_W1, r_b1, r_W2, r_b2):
    src2 = jnp.concatenate(
        [edge_index[0], jnp.zeros((E2 - E,), jnp.int32)])
    dst2 = jnp.concatenate(
        [edge_index[1], jnp.full((E2 - E,), N, jnp.int32)])
    pos16 = jnp.pad(pos, ((0, NPAD - N), (0, 13)))

    x = _encode(h, W_enc, b_enc)
    x = _conv(x, pos16, src2, dst2, h1_W1, h1_b1, h1_W2, h1_b2,
              f1_W1, f1_b1, f1_W2, f1_b2, g1_W1, g1_b1, g1_gamma, g1_beta,
              g1_W2, g1_b2)
    x = _conv(x, pos16, src2, dst2, h2_W1, h2_b1, h2_W2, h2_b2,
              f2_W1, f2_b1, f2_W2, f2_b2, g2_W1, g2_b1, g2_gamma, g2_beta,
              g2_W2, g2_b2)
    return _pool_head(x, batch, r_W1, r_b1, r_W2, r_b2)
